# baseline (speedup 1.0000x reference)
import functools

import jax
import jax.numpy as jnp
from jax import lax
from jax.experimental import pallas as pl
from jax.experimental.pallas import tpu as pltpu
from jax.experimental.pallas import tpu_sc as plsc

N_NODES = 10000
D = 128
G = 128
NC = 2
NS = 16
NW = NC * NS
SUB = 104
NSUB = 2
RPW = SUB * NSUB
SC_ROWS = NW * RPW
TC_ROWS = N_NODES - SC_ROWS

_mesh = plsc.VectorSubcoreMesh(core_axis_name="c", subcore_axis_name="s")


@functools.partial(
    pl.kernel,
    mesh=_mesh,
    out_type=jax.ShapeDtypeStruct((NC, G, D), jnp.float32),
    scratch_types=[
        pltpu.VMEM((RPW, D), jnp.float32),
        pltpu.VMEM((NSUB, SUB), jnp.int32),
        pltpu.VMEM_SHARED((G, D), jnp.float32),
        pltpu.SemaphoreType.DMA,
        pltpu.SemaphoreType.DMA,
        pltpu.SemaphoreType.DMA,
        pltpu.SemaphoreType.DMA,
        pltpu.SemaphoreType.DMA,
    ],
)
def _segment_sum_sc(x_hbm, batch_hbm, zeros_hbm, out_hbm, xbuf, idxbuf,
                    acc, sem_i, sem_x0, sem_x1, sem_s, sem_z):
    cid = lax.axis_index("c")
    sid = lax.axis_index("s")
    w = cid * NS + sid
    rpt = G // NS
    sems_x = (sem_x0, sem_x1)

    c_z = pltpu.async_copy(zeros_hbm.at[pl.ds(sid * rpt, rpt)],
                           acc.at[pl.ds(sid * rpt, rpt)], sem_z)

    base = w * RPW
    c_ids = [
        pltpu.async_copy(batch_hbm.at[pl.ds(base + j * SUB, SUB)],
                         idxbuf.at[j], sem_i)
        for j in range(NSUB)
    ]
    c_x = [
        pltpu.async_copy(x_hbm.at[pl.ds(base + j * SUB, SUB)],
                         xbuf.at[pl.ds(j * SUB, SUB)], sems_x[j])
        for j in range(NSUB)
    ]

    c_z.wait()
    plsc.subcore_barrier()

    for c in c_ids:
        c.wait()
    scats = []
    for j in range(NSUB):
        c_x[j].wait()
        scats.append(
            pltpu.async_copy(xbuf.at[pl.ds(j * SUB, SUB)],
                             acc.at[idxbuf.at[j]], sem_s, add=True))
    for c in scats:
        c.wait()
    plsc.subcore_barrier()

    pltpu.sync_copy(acc.at[pl.ds(sid * rpt, rpt)],
                    out_hbm.at[cid, pl.ds(sid * rpt, rpt)])


def _tc_pool_body(x_ref, ids_ref, out_ref):
    ids = ids_ref[...]
    gids = lax.broadcasted_iota(jnp.int32, (G, TC_ROWS), 0)
    onehot = (gids == ids).astype(jnp.float32)
    out_ref[...] = jnp.dot(onehot, x_ref[...],
                           precision=lax.Precision.HIGHEST,
                           preferred_element_type=jnp.float32)


def _mlp_body(parts_ref, part_tc_ref, w1_ref, b1_ref, w2_ref, b2_ref,
              out_ref):
    pooled = parts_ref[0] + parts_ref[1] + part_tc_ref[...]
    h = jnp.dot(pooled, w1_ref[...], preferred_element_type=jnp.float32)
    h = jnp.maximum(h + b1_ref[...], 0.0)
    out_t = lax.dot_general(w2_ref[...], h, (((0,), (1,)), ((), ())),
                            preferred_element_type=jnp.float32)
    out_ref[...] = out_t + b2_ref[...]


def kernel(x, edge_index, batch, W1, b1, W2, b2):
    del edge_index
    zeros = jnp.zeros((G, D), jnp.float32)
    parts = _segment_sum_sc(x, batch, zeros)
    part_tc = pl.pallas_call(
        _tc_pool_body,
        out_shape=jax.ShapeDtypeStruct((G, D), jnp.float32),
    )(x[SC_ROWS:], batch[SC_ROWS:].reshape(1, -1))
    out_t = pl.pallas_call(
        _mlp_body,
        out_shape=jax.ShapeDtypeStruct((W2.shape[1], G), jnp.float32),
    )(parts, part_tc, W1, b1.reshape(1, -1), W2, b2.reshape(-1, 1))
    return out_t.T


# speedup vs baseline: 1.0180x; 1.0180x over previous
import functools

import jax
import jax.numpy as jnp
from jax import lax
from jax.experimental import pallas as pl
from jax.experimental.pallas import tpu as pltpu
from jax.experimental.pallas import tpu_sc as plsc

N_NODES = 10000
D = 128
G = 128
NC = 2
NS = 16
NW = NC * NS
SUB = 104
NSUB = 3
RPW = SUB * NSUB
SC_ROWS = NW * RPW
TC_ROWS = N_NODES - SC_ROWS

_mesh = plsc.VectorSubcoreMesh(core_axis_name="c", subcore_axis_name="s")


@functools.partial(
    pl.kernel,
    mesh=_mesh,
    out_type=jax.ShapeDtypeStruct((NC, G, D), jnp.float32),
    scratch_types=[
        pltpu.VMEM((RPW, D), jnp.float32),
        pltpu.VMEM((NSUB, SUB), jnp.int32),
        pltpu.VMEM_SHARED((G, D), jnp.float32),
        pltpu.SemaphoreType.DMA,
        pltpu.SemaphoreType.DMA,
        pltpu.SemaphoreType.DMA,
        pltpu.SemaphoreType.DMA,
        pltpu.SemaphoreType.DMA,
        pltpu.SemaphoreType.DMA,
    ],
)
def _segment_sum_sc(x_hbm, batch_hbm, zeros_hbm, out_hbm, xbuf, idxbuf,
                    acc, sem_i, sem_x0, sem_x1, sem_x2, sem_s, sem_z):
    cid = lax.axis_index("c")
    sid = lax.axis_index("s")
    w = cid * NS + sid
    rpt = G // NS
    sems_x = (sem_x0, sem_x1, sem_x2)

    c_z = pltpu.async_copy(zeros_hbm.at[pl.ds(sid * rpt, rpt)],
                           acc.at[pl.ds(sid * rpt, rpt)], sem_z)

    base = w * RPW
    c_ids = [
        pltpu.async_copy(batch_hbm.at[pl.ds(base + j * SUB, SUB)],
                         idxbuf.at[j], sem_i)
        for j in range(NSUB)
    ]
    c_x = [
        pltpu.async_copy(x_hbm.at[pl.ds(base + j * SUB, SUB)],
                         xbuf.at[pl.ds(j * SUB, SUB)], sems_x[j])
        for j in range(NSUB)
    ]

    c_z.wait()
    plsc.subcore_barrier()

    for c in c_ids:
        c.wait()
    scats = []
    for j in range(NSUB):
        c_x[j].wait()
        scats.append(
            pltpu.async_copy(xbuf.at[pl.ds(j * SUB, SUB)],
                             acc.at[idxbuf.at[j]], sem_s, add=True))
    for c in scats:
        c.wait()
    plsc.subcore_barrier()

    pltpu.sync_copy(acc.at[pl.ds(sid * rpt, rpt)],
                    out_hbm.at[cid, pl.ds(sid * rpt, rpt)])


def _tc_pool_body(x_ref, ids_ref, out_ref):
    ids = ids_ref[...]
    gids = lax.broadcasted_iota(jnp.int32, (G, TC_ROWS), 0)
    onehot = (gids == ids).astype(jnp.float32)
    out_ref[...] = jnp.dot(onehot, x_ref[...],
                           precision=lax.Precision.HIGHEST,
                           preferred_element_type=jnp.float32)


def _mlp_body(parts_ref, part_tc_ref, w1_ref, b1_ref, w2_ref, b2_ref,
              out_ref):
    pooled = parts_ref[0] + parts_ref[1] + part_tc_ref[...]
    h = jnp.dot(pooled, w1_ref[...], preferred_element_type=jnp.float32)
    h = jnp.maximum(h + b1_ref[...], 0.0)
    out_t = lax.dot_general(w2_ref[...], h, (((0,), (1,)), ((), ())),
                            preferred_element_type=jnp.float32)
    out_ref[...] = out_t + b2_ref[...]


def kernel(x, edge_index, batch, W1, b1, W2, b2):
    del edge_index
    zeros = jnp.zeros((G, D), jnp.float32)
    parts = _segment_sum_sc(x, batch, zeros)
    part_tc = pl.pallas_call(
        _tc_pool_body,
        out_shape=jax.ShapeDtypeStruct((G, D), jnp.float32),
    )(x[SC_ROWS:], batch[SC_ROWS:].reshape(1, -1))
    out_t = pl.pallas_call(
        _mlp_body,
        out_shape=jax.ShapeDtypeStruct((W2.shape[1], G), jnp.float32),
    )(parts, part_tc, W1, b1.reshape(1, -1), W2, b2.reshape(-1, 1))
    return out_t.T
